# baseline (speedup 1.0000x reference)
import jax
import jax.numpy as jnp
from jax import lax
from jax.experimental import pallas as pl
from jax.experimental.pallas import tpu as pltpu
from jax.experimental.pallas import tpu_sc as plsc

N = 10000
E = 320000
D = 128
FC = 256
NC = 40
G = 16

NPAD = 10240
NTILES = 32
RPT = NPAD // 16

DCHUNK = 128
DCPT = 80
DEPAD = NTILES * DCPT * DCHUNK
KD = 8

SCHUNK = 128
SCPT = 80
SEPAD = NTILES * SCPT * SCHUNK
KS = 2
NW = 8
NWIN = SCPT // NW

_MESH = plsc.VectorSubcoreMesh(core_axis_name="c", subcore_axis_name="s")
_f32 = jnp.float32


def _deg_body(col_hbm, ones_hbm, zd_hbm, out_hbm, idx_v, ones_v, acc_sh, sem):
    c = lax.axis_index("c")
    s = lax.axis_index("s")
    wid = c * 16 + s
    sl = pl.ds(s * RPT, RPT)
    pltpu.sync_copy(zd_hbm.at[sl], acc_sh.at[sl])
    pltpu.sync_copy(ones_hbm, ones_v)
    pltpu.sync_copy(col_hbm.at[wid], idx_v)
    plsc.subcore_barrier()

    @pl.loop(0, DCPT // KD)
    def _(g):
        descs = [
            pltpu.async_copy(ones_v, acc_sh.at[idx_v.at[g * KD + p]], sem,
                             add=True)
            for p in range(KD)
        ]
        for desc in descs:
            desc.wait()

    plsc.subcore_barrier()
    pltpu.sync_copy(acc_sh.at[sl], out_hbm.at[c, sl])


_deg_call = pl.kernel(
    _deg_body,
    out_type=jax.ShapeDtypeStruct((2, NPAD, D), _f32),
    mesh=_MESH,
    scratch_types=[
        pltpu.VMEM((DCPT, DCHUNK), jnp.int32),
        pltpu.VMEM((DCHUNK, D), _f32),
        pltpu.VMEM_SHARED((NPAD, D), _f32),
        pltpu.SemaphoreType.DMA,
    ],
)


def _scat_body(h_hbm, row_hbm, col_hbm, zd_hbm, out_hbm,
               rid_v, cid_v, tmp_v, acc_sh, gsem, ssem, isem):
    c = lax.axis_index("c")
    s = lax.axis_index("s")
    wid = c * 16 + s
    sl = pl.ds(s * RPT, RPT)
    pltpu.sync_copy(zd_hbm.at[sl], acc_sh.at[sl])
    pltpu.async_copy(row_hbm.at[wid, pl.ds(0, NW)], rid_v.at[0], isem)
    pltpu.async_copy(col_hbm.at[wid, pl.ds(0, NW)], cid_v.at[0], isem)
    plsc.subcore_barrier()

    @pl.loop(0, NWIN)
    def _(w):
        wb = w % 2

        @pl.when(w + 1 < NWIN)
        def _():
            nb = (w + 1) % 2
            nsl = pl.ds((w + 1) * NW, NW)
            pltpu.async_copy(row_hbm.at[wid, nsl], rid_v.at[nb], isem)
            pltpu.async_copy(col_hbm.at[wid, nsl], cid_v.at[nb], isem)

        wsl = pl.ds(w * NW, NW)
        pltpu.make_async_copy(row_hbm.at[wid, wsl], rid_v.at[wb], isem).wait()
        pltpu.make_async_copy(col_hbm.at[wid, wsl], cid_v.at[wb], isem).wait()

        @pl.loop(0, NW // KS)
        def _(g):
            gd = [
                pltpu.async_copy(h_hbm.at[rid_v.at[wb, g * KS + p]],
                                 tmp_v.at[p], gsem)
                for p in range(KS)
            ]
            sd = []
            for p in range(KS):
                gd[p].wait()
                sd.append(
                    pltpu.async_copy(tmp_v.at[p],
                                     acc_sh.at[cid_v.at[wb, g * KS + p]],
                                     ssem, add=True))
            for desc in sd:
                desc.wait()

    plsc.subcore_barrier()
    pltpu.sync_copy(acc_sh.at[sl], out_hbm.at[c, sl])


_scat_call = pl.kernel(
    _scat_body,
    out_type=jax.ShapeDtypeStruct((2, NPAD, D), _f32),
    mesh=_MESH,
    scratch_types=[
        pltpu.VMEM((2, NW, SCHUNK), jnp.int32),
        pltpu.VMEM((2, NW, SCHUNK), jnp.int32),
        pltpu.VMEM((KS, SCHUNK, D), _f32),
        pltpu.VMEM_SHARED((NPAD, D), _f32),
        pltpu.SemaphoreType.DMA,
        pltpu.SemaphoreType.DMA,
        pltpu.SemaphoreType.DMA,
    ],
)


def _mm0_body(x_ref, w_ref, o_ref):
    o_ref[...] = jnp.dot(x_ref[...], w_ref[...], preferred_element_type=_f32)


_mm0_call = pl.pallas_call(
    _mm0_body, out_shape=jax.ShapeDtypeStruct((NPAD, D), _f32))


def _scale0_body(xw_ref, deg_ref, hs_ref, dis_ref):
    d = deg_ref[0][:, 0:1] + deg_ref[1][:, 0:1]
    dis = jnp.where(d > 0, lax.rsqrt(d), 0.0)
    dis_ref[...] = dis
    hs_ref[...] = xw_ref[...] * dis


_scale0_call = pl.pallas_call(
    _scale0_body,
    out_shape=(jax.ShapeDtypeStruct((NPAD, D), _f32),
               jax.ShapeDtypeStruct((NPAD, 1), _f32)))


def _mid_body(acc_ref, dis_ref, b_ref, g_ref, be_ref, w_ref, o_ref):
    dis = dis_ref[...]
    h = (acc_ref[0] + acc_ref[1]) * dis + b_ref[...]
    hr = h[0:N]
    m = jnp.mean(hr, axis=0, keepdims=True)
    v = jnp.mean((hr - m) ** 2, axis=0, keepdims=True)
    hn = (h - m) * lax.rsqrt(v + 1e-5) * g_ref[...] + be_ref[...]
    hn = jnp.maximum(hn, 0.0)
    o_ref[...] = jnp.dot(hn, w_ref[...], preferred_element_type=_f32) * dis


_mid_call = pl.pallas_call(
    _mid_body, out_shape=jax.ShapeDtypeStruct((NPAD, D), _f32))


def _fin_body(acc_ref, dis_ref, b_ref, g_ref, be_ref, batch_ref,
              w1_ref, b1_ref, w2_ref, b2_ref, o_ref):
    h = (acc_ref[0] + acc_ref[1]) * dis_ref[...] + b_ref[...]
    hr = h[0:N]
    m = jnp.mean(hr, axis=0, keepdims=True)
    v = jnp.mean((hr - m) ** 2, axis=0, keepdims=True)
    hn = (hr - m) * lax.rsqrt(v + 1e-5) * g_ref[...] + be_ref[...]
    hn = jnp.maximum(hn, 0.0)
    oh = (batch_ref[...] ==
          lax.broadcasted_iota(jnp.int32, (G, N), 0)).astype(_f32)
    pooled = jnp.dot(oh, hn, preferred_element_type=_f32)
    cnt = jnp.dot(oh, jnp.ones((N, 1), _f32), preferred_element_type=_f32)
    pooled = pooled / jnp.maximum(cnt, 1.0)
    z = jnp.maximum(
        jnp.dot(pooled, w1_ref[...], preferred_element_type=_f32) + b1_ref[...],
        0.0)
    o_ref[...] = jnp.dot(z, w2_ref[...], preferred_element_type=_f32) + b2_ref[...]


_fin_call = pl.pallas_call(
    _fin_body, out_shape=jax.ShapeDtypeStruct((G, NC), _f32))


def kernel(x, edge_index, batch, W0, b0, g0, be0, W1, b1, g1, be1,
           fc1_W, fc1_b, fc2_W, fc2_b):
    dpad = jnp.full((DEPAD - E,), N, jnp.int32)
    spad = jnp.full((SEPAD - E,), N, jnp.int32)
    col_d = jnp.concatenate([edge_index[1], dpad]).reshape(NTILES, DCPT, DCHUNK)
    row_s = jnp.concatenate([edge_index[0], spad]).reshape(NTILES, SCPT, SCHUNK)
    col_s = jnp.concatenate([edge_index[1], spad]).reshape(NTILES, SCPT, SCHUNK)
    x_pad = jnp.pad(x, ((0, NPAD - N), (0, 0)))
    zD = jnp.zeros((NPAD, D), _f32)
    ones_c = jnp.ones((DCHUNK, D), _f32)

    degp = _deg_call(col_d, ones_c, zD)
    xw = _mm0_call(x_pad, W0)
    hs0, dis = _scale0_call(xw, degp)
    acc1 = _scat_call(hs0, row_s, col_s, zD)
    hs1 = _mid_call(acc1, dis, b0.reshape(1, D), g0.reshape(1, D),
                    be0.reshape(1, D), W1)
    acc2 = _scat_call(hs1, row_s, col_s, zD)
    out = _fin_call(acc2, dis, b1.reshape(1, D), g1.reshape(1, D),
                    be1.reshape(1, D), batch.reshape(1, N),
                    fc1_W, fc1_b.reshape(1, FC), fc2_W, fc2_b.reshape(1, NC))
    return out


# speedup vs baseline: 2.5804x; 2.5804x over previous
import jax
import jax.numpy as jnp
from jax import lax
from jax.experimental import pallas as pl
from jax.experimental.pallas import tpu as pltpu
from jax.experimental.pallas import tpu_sc as plsc

N = 10000
E = 320000
D = 128
FC = 256
NC = 40
G = 16

NPAD = 10240
NTILES = 32
RPT = NPAD // 16

DCHUNK = 128
DCPT = 80
DEPAD = NTILES * DCPT * DCHUNK
KD = 8

SCHUNK = 128
SCPT = 80
SEPAD = NTILES * SCPT * SCHUNK
KS = 2
NW = 8
NWIN = SCPT // NW

_MESH = plsc.VectorSubcoreMesh(core_axis_name="c", subcore_axis_name="s")
_f32 = jnp.float32


def _deg_body(col_hbm, ones_hbm, zd_hbm, out_hbm, idx_v, ones_v, acc_sh, sem):
    c = lax.axis_index("c")
    s = lax.axis_index("s")
    wid = c * 16 + s
    sl = pl.ds(s * RPT, RPT)
    pltpu.sync_copy(zd_hbm.at[sl], acc_sh.at[sl])
    pltpu.sync_copy(ones_hbm, ones_v)
    pltpu.sync_copy(col_hbm.at[wid], idx_v)
    plsc.subcore_barrier()

    @pl.loop(0, DCPT // KD)
    def _(g):
        descs = [
            pltpu.async_copy(ones_v, acc_sh.at[idx_v.at[g * KD + p]], sem,
                             add=True)
            for p in range(KD)
        ]
        for desc in descs:
            desc.wait()

    plsc.subcore_barrier()
    pltpu.sync_copy(acc_sh.at[sl], out_hbm.at[c, sl])


_deg_call = pl.kernel(
    _deg_body,
    out_type=jax.ShapeDtypeStruct((2, NPAD, D), _f32),
    mesh=_MESH,
    scratch_types=[
        pltpu.VMEM((DCPT, DCHUNK), jnp.int32),
        pltpu.VMEM((DCHUNK, D), _f32),
        pltpu.VMEM_SHARED((NPAD, D), _f32),
        pltpu.SemaphoreType.DMA,
    ],
)


def _scat_body(h_hbm, row_hbm, col_hbm, zd_hbm, out_hbm,
               rid_v, cid_v, tmp_v, acc_sh, gsem, ssem, isem):
    c = lax.axis_index("c")
    s = lax.axis_index("s")
    wid = c * 16 + s
    sl = pl.ds(s * RPT, RPT)
    pltpu.sync_copy(zd_hbm.at[sl], acc_sh.at[sl])
    pltpu.async_copy(row_hbm.at[wid, pl.ds(0, NW)], rid_v.at[0], isem)
    pltpu.async_copy(col_hbm.at[wid, pl.ds(0, NW)], cid_v.at[0], isem)
    plsc.subcore_barrier()

    @pl.loop(0, NWIN)
    def _(w):
        wb = w % 2

        @pl.when(w + 1 < NWIN)
        def _():
            nb = (w + 1) % 2
            nsl = pl.ds((w + 1) * NW, NW)
            pltpu.async_copy(row_hbm.at[wid, nsl], rid_v.at[nb], isem)
            pltpu.async_copy(col_hbm.at[wid, nsl], cid_v.at[nb], isem)

        wsl = pl.ds(w * NW, NW)
        pltpu.make_async_copy(row_hbm.at[wid, wsl], rid_v.at[wb], isem).wait()
        pltpu.make_async_copy(col_hbm.at[wid, wsl], cid_v.at[wb], isem).wait()

        @pl.loop(0, NW // KS)
        def _(g):
            gd = [
                pltpu.async_copy(h_hbm.at[rid_v.at[wb, g * KS + p]],
                                 tmp_v.at[p], gsem)
                for p in range(KS)
            ]
            sd = []
            for p in range(KS):
                gd[p].wait()
                sd.append(
                    pltpu.async_copy(tmp_v.at[p],
                                     acc_sh.at[cid_v.at[wb, g * KS + p]],
                                     ssem, add=True))
            for desc in sd:
                desc.wait()

    plsc.subcore_barrier()
    pltpu.sync_copy(acc_sh.at[sl], out_hbm.at[c, sl])


_scat_call = pl.kernel(
    _scat_body,
    out_type=jax.ShapeDtypeStruct((2, NPAD, D), _f32),
    mesh=_MESH,
    scratch_types=[
        pltpu.VMEM((2, NW, SCHUNK), jnp.int32),
        pltpu.VMEM((2, NW, SCHUNK), jnp.int32),
        pltpu.VMEM((KS, SCHUNK, D), _f32),
        pltpu.VMEM_SHARED((NPAD, D), _f32),
        pltpu.SemaphoreType.DMA,
        pltpu.SemaphoreType.DMA,
        pltpu.SemaphoreType.DMA,
    ],
)


def _mm0_body(x_ref, w_ref, o_ref):
    o_ref[...] = jnp.dot(x_ref[...], w_ref[...], preferred_element_type=_f32)


_mm0_call = pl.pallas_call(
    _mm0_body, out_shape=jax.ShapeDtypeStruct((NPAD, D), _f32))


def _scale0_body(xw_ref, deg_ref, hs_ref, dis_ref):
    d = deg_ref[0][:, 0:1] + deg_ref[1][:, 0:1]
    dis = jnp.where(d > 0, lax.rsqrt(d), 0.0)
    dis_ref[...] = dis
    hs_ref[...] = xw_ref[...] * dis


_scale0_call = pl.pallas_call(
    _scale0_body,
    out_shape=(jax.ShapeDtypeStruct((NPAD, D), _f32),
               jax.ShapeDtypeStruct((NPAD, 1), _f32)))


def _mid_body(acc_ref, dis_ref, b_ref, g_ref, be_ref, w_ref, o_ref):
    dis = dis_ref[...]
    h = (acc_ref[0] + acc_ref[1]) * dis + b_ref[...]
    hr = h[0:N]
    m = jnp.mean(hr, axis=0, keepdims=True)
    v = jnp.mean((hr - m) ** 2, axis=0, keepdims=True)
    hn = (h - m) * lax.rsqrt(v + 1e-5) * g_ref[...] + be_ref[...]
    hn = jnp.maximum(hn, 0.0)
    o_ref[...] = jnp.dot(hn, w_ref[...], preferred_element_type=_f32) * dis


_mid_call = pl.pallas_call(
    _mid_body, out_shape=jax.ShapeDtypeStruct((NPAD, D), _f32))


def _fin_body(acc_ref, dis_ref, b_ref, g_ref, be_ref, batch_ref,
              w1_ref, b1_ref, w2_ref, b2_ref, o_ref):
    h = (acc_ref[0] + acc_ref[1]) * dis_ref[...] + b_ref[...]
    hr = h[0:N]
    m = jnp.mean(hr, axis=0, keepdims=True)
    v = jnp.mean((hr - m) ** 2, axis=0, keepdims=True)
    hn = (hr - m) * lax.rsqrt(v + 1e-5) * g_ref[...] + be_ref[...]
    hn = jnp.maximum(hn, 0.0)
    oh = (batch_ref[...] ==
          lax.broadcasted_iota(jnp.int32, (G, N), 0)).astype(_f32)
    pooled = jnp.dot(oh, hn, preferred_element_type=_f32)
    cnt = jnp.dot(oh, jnp.ones((N, 1), _f32), preferred_element_type=_f32)
    pooled = pooled / jnp.maximum(cnt, 1.0)
    z = jnp.maximum(
        jnp.dot(pooled, w1_ref[...], preferred_element_type=_f32) + b1_ref[...],
        0.0)
    o_ref[...] = jnp.dot(z, w2_ref[...], preferred_element_type=_f32) + b2_ref[...]


_fin_call = pl.pallas_call(
    _fin_body, out_shape=jax.ShapeDtypeStruct((G, NC), _f32))


def kernel(x, edge_index, batch, W0, b0, g0, be0, W1, b1, g1, be1,
           fc1_W, fc1_b, fc2_W, fc2_b):
    dpad = N + jnp.arange(DEPAD - E, dtype=jnp.int32) % (NPAD - N)
    spad = N + jnp.arange(SEPAD - E, dtype=jnp.int32) % (NPAD - N)
    col_d = jnp.concatenate([edge_index[1], dpad]).reshape(NTILES, DCPT, DCHUNK)
    row_s = jnp.concatenate([edge_index[0], spad]).reshape(NTILES, SCPT, SCHUNK)
    col_s = jnp.concatenate([edge_index[1], spad]).reshape(NTILES, SCPT, SCHUNK)
    x_pad = jnp.pad(x, ((0, NPAD - N), (0, 0)))
    zD = jnp.zeros((NPAD, D), _f32)
    ones_c = jnp.ones((DCHUNK, D), _f32)

    degp = _deg_call(col_d, ones_c, zD)
    xw = _mm0_call(x_pad, W0)
    hs0, dis = _scale0_call(xw, degp)
    acc1 = _scat_call(hs0, row_s, col_s, zD)
    hs1 = _mid_call(acc1, dis, b0.reshape(1, D), g0.reshape(1, D),
                    be0.reshape(1, D), W1)
    acc2 = _scat_call(hs1, row_s, col_s, zD)
    out = _fin_call(acc2, dis, b1.reshape(1, D), g1.reshape(1, D),
                    be1.reshape(1, D), batch.reshape(1, N),
                    fc1_W, fc1_b.reshape(1, FC), fc2_W, fc2_b.reshape(1, NC))
    return out
